# baseline (speedup 1.0000x reference)
import functools

import jax
import jax.numpy as jnp
from jax import lax
from jax.experimental import pallas as pl
from jax.experimental.pallas import tpu as pltpu
from jax.experimental.pallas import tpu_sc as plsc

_D = 64
_DP = 588
_K = 1024
_S = 256
_B = 4
_H = 4
_NB = 2
_DFF = 256
_BS = _B * _S
_DH = _D // _H
_TQ = 128
_NQ = _BS // _TQ

_NC = 2
_NS = 16
_NW = _NC * _NS
_RPW = _BS // _NW


def _layernorm(x, s, b):
    m = jnp.mean(x, axis=-1, keepdims=True)
    v = jnp.mean((x - m) * (x - m), axis=-1, keepdims=True)
    return (x - m) / jnp.sqrt(v + 1e-5) * s[None, :] + b[None, :]


def _gelu_tanh(x):
    c = 0.7978845608028654
    return 0.5 * x * (1.0 + jnp.tanh(c * (x + 0.044715 * (x * x * x))))


def _dot(a, b, dims, prec=jax.lax.Precision.DEFAULT):
    return jax.lax.dot_general(a, b, (dims, ((), ())), precision=prec,
                               preferred_element_type=jnp.float32)


_HI = jax.lax.Precision.HIGHEST


def fused_body(zef_ref, cb_ref, pos_ref, ln1s_ref, ln1b_ref, wqkv_ref,
               bqkv_ref, wo_ref, bo_ref, ln2s_ref, ln2b_ref, w1_ref, b1_ref,
               w2_ref, b2_ref, lnfs_ref, lnfb_ref, wout_ref, bout_ref,
               logits_ref, idx_ref, zq_s):
    pid = pl.program_id(0)

    @pl.when(pid < _NQ)
    def _vq():
        zef = zef_ref[...]
        cb = cb_ref[...]
        cn_col = jnp.sum(cb * cb, axis=1, keepdims=True)
        a_aug = jnp.concatenate(
            [zef * -2.0, jnp.ones((_TQ, 1), jnp.float32)], axis=1)
        b_aug = jnp.concatenate([cb, cn_col], axis=1)
        scores = _dot(a_aug, b_aug, (((1,), (1,))), _HI)
        mn = jnp.min(scores, axis=1, keepdims=True)
        kiota = jax.lax.broadcasted_iota(jnp.int32, (_TQ, _K), 1)
        idx = jnp.min(jnp.where(scores <= mn, kiota, _K), axis=1,
                      keepdims=True)
        idx_ref[...] = idx
        onehot = (kiota == idx).astype(jnp.float32)
        zq = _dot(onehot, cb, (((1,), (0,))), _HI)
        zq_s[pl.ds(pid * _TQ, _TQ), :] = zq

    @pl.when(pid >= _NQ)
    def _vit():
        b = pid - _NQ
        x = zq_s[pl.ds(b * _S, _S), :] + pos_ref[...]
        head_of_lane = jax.lax.broadcasted_iota(jnp.int32, (_S, _D), 1) // _DH

        for i in range(_NB):
            h = _layernorm(x, ln1s_ref[i], ln1b_ref[i])
            qkv = _dot(h, wqkv_ref[i], (((1,), (0,)))) + bqkv_ref[i][None, :]
            q = qkv[:, 0:_D]
            k = qkv[:, _D:2 * _D]
            v = qkv[:, 2 * _D:3 * _D]
            qm = jnp.concatenate(
                [jnp.where(head_of_lane == hh, q, 0.0) for hh in range(_H)],
                axis=0)
            al = _dot(qm, k, (((1,), (1,)))) * 0.25
            al = al - jnp.max(al, axis=1, keepdims=True)
            e = jnp.exp(al)
            p = e / jnp.sum(e, axis=1, keepdims=True)
            ost = _dot(p, v, (((1,), (0,))))
            o = jnp.zeros((_S, _D), jnp.float32)
            for hh in range(_H):
                o = o + jnp.where(head_of_lane == hh,
                                  ost[hh * _S:(hh + 1) * _S], 0.0)
            x = x + _dot(o, wo_ref[i], (((1,), (0,)))) + bo_ref[i][None, :]
            h2 = _layernorm(x, ln2s_ref[i], ln2b_ref[i])
            g = _dot(h2, w1_ref[i], (((1,), (0,)))) + b1_ref[i][None, :]
            x = x + _dot(_gelu_tanh(g), w2_ref[i], (((1,), (0,)))) \
                + b2_ref[i][None, :]

        xf = _layernorm(x, lnfs_ref[...], lnfb_ref[...])
        logits_ref[...] = _dot(xf, wout_ref[...], (((1,), (0,)))) \
            + bout_ref[...][None, :]


def sc_gather_body(cb_hbm, idx_hbm, out_hbm, idx_v, rows_v, sem):
    wid = lax.axis_index("s") * _NC + lax.axis_index("c")
    base = wid * _RPW
    pltpu.sync_copy(idx_hbm.at[pl.ds(base, _RPW)], idx_v)
    pltpu.async_copy(cb_hbm.at[idx_v], rows_v, sem).wait()
    pltpu.sync_copy(rows_v, out_hbm.at[pl.ds(base, _RPW)])


_sc_gather = functools.partial(
    pl.kernel,
    out_type=jax.ShapeDtypeStruct((_BS, _D), jnp.float32),
    mesh=plsc.VectorSubcoreMesh(core_axis_name="c", subcore_axis_name="s"),
    compiler_params=pltpu.CompilerParams(use_tc_tiling_on_sc=False),
    scratch_types=[
        pltpu.VMEM((_RPW,), jnp.int32),
        pltpu.VMEM((_RPW, _D), jnp.float32),
        pltpu.SemaphoreType.DMA,
    ],
)(sc_gather_body)


def _full(shape):
    return pl.BlockSpec(shape, lambda i: tuple(0 for _ in shape))


@jax.jit
def _run(zef, codebook, pos_emb, ln1_s, ln1_b, Wqkv, bqkv, Wo, bo, ln2_s,
         ln2_b, W1, b1, W2, b2, lnf_s, lnf_b, Wout, bout):
    logits, idx = pl.pallas_call(
        fused_body,
        grid=(_NQ + _B,),
        in_specs=[
            pl.BlockSpec((_TQ, _D), lambda i: (jnp.minimum(i, _NQ - 1), 0)),
            _full((_K, _D)),
            _full((_S, _D)),
            _full((_NB, _D)), _full((_NB, _D)),
            _full((_NB, _D, 3 * _D)), _full((_NB, 3 * _D)),
            _full((_NB, _D, _D)), _full((_NB, _D)),
            _full((_NB, _D)), _full((_NB, _D)),
            _full((_NB, _D, _DFF)), _full((_NB, _DFF)),
            _full((_NB, _DFF, _D)), _full((_NB, _D)),
            _full((_D,)), _full((_D,)),
            _full((_D, _DP)), _full((_DP,)),
        ],
        out_specs=(
            pl.BlockSpec((_S, _DP), lambda i: (jnp.maximum(i - _NQ, 0), 0)),
            pl.BlockSpec((_TQ, 1), lambda i: (jnp.minimum(i, _NQ - 1), 0)),
        ),
        out_shape=(
            jax.ShapeDtypeStruct((_BS, _DP), jnp.float32),
            jax.ShapeDtypeStruct((_BS, 1), jnp.int32),
        ),
        scratch_shapes=[pltpu.VMEM((_BS, _D), jnp.float32)],
    )(zef, codebook, pos_emb, ln1_s, ln1_b, Wqkv, bqkv, Wo, bo, ln2_s,
      ln2_b, W1, b1, W2, b2, lnf_s, lnf_b, Wout, bout)
    zq = _sc_gather(codebook, idx.reshape(_BS))
    return logits, zq


def kernel(ze, codebook, pos_emb, ln1_s, ln1_b, Wqkv, bqkv, Wo, bo, ln2_s,
           ln2_b, W1, b1, W2, b2, lnf_s, lnf_b, Wout, bout):
    zef = ze.reshape(_BS, _D)
    logits, zq = _run(zef, codebook, pos_emb, ln1_s, ln1_b, Wqkv, bqkv, Wo,
                      bo, ln2_s, ln2_b, W1, b1, W2, b2, lnf_s, lnf_b, Wout,
                      bout)
    return logits.reshape(_B, _S, _DP), zq.reshape(_B, _S, _D)


# speedup vs baseline: 1.0220x; 1.0220x over previous
import functools

import jax
import jax.numpy as jnp
from jax import lax
from jax.experimental import pallas as pl
from jax.experimental.pallas import tpu as pltpu
from jax.experimental.pallas import tpu_sc as plsc

_D = 64
_DP = 588
_K = 1024
_S = 256
_B = 4
_H = 4
_NB = 2
_DFF = 256
_BS = _B * _S
_DH = _D // _H
_TQ = 128
_NQ = _BS // _TQ

_NC = 2
_NS = 16
_NW = _NC * _NS
_RPW = _BS // _NW


def _layernorm(x, s, b):
    m = jnp.mean(x, axis=-1, keepdims=True)
    v = jnp.mean((x - m) * (x - m), axis=-1, keepdims=True)
    return (x - m) / jnp.sqrt(v + 1e-5) * s[None, :] + b[None, :]


def _gelu_tanh(x):
    c = 0.7978845608028654
    return 0.5 * x * (1.0 + jnp.tanh(c * (x + 0.044715 * (x * x * x))))


def _dot(a, b, dims, prec=jax.lax.Precision.DEFAULT):
    return jax.lax.dot_general(a, b, (dims, ((), ())), precision=prec,
                               preferred_element_type=jnp.float32)


_HI = jax.lax.Precision.HIGHEST


def vq_body(zef_ref, cb_ref, zq_ref, idx_ref):
    if True:
        zef = zef_ref[...]
        cb = cb_ref[...]
        cn_col = jnp.sum(cb * cb, axis=1, keepdims=True)
        a_aug = jnp.concatenate(
            [zef * -2.0, jnp.ones((_TQ, 1), jnp.float32)], axis=1)
        b_aug = jnp.concatenate([cb, cn_col], axis=1)
        scores = _dot(a_aug, b_aug, (((1,), (1,))), _HI)
        mn = jnp.min(scores, axis=1, keepdims=True)
        kiota = jax.lax.broadcasted_iota(jnp.int32, (_TQ, _K), 1)
        idx = jnp.min(jnp.where(scores <= mn, kiota, _K), axis=1,
                      keepdims=True)
        idx_ref[...] = idx
        onehot = (kiota == idx).astype(jnp.float32)
        zq_ref[...] = _dot(onehot, cb, (((1,), (0,))), _HI)


def vit_body(zq_ref, pos_ref, ln1s_ref, ln1b_ref, wqkv_ref, bqkv_ref,
             wo_ref, bo_ref, ln2s_ref, ln2b_ref, w1_ref, b1_ref, w2_ref,
             b2_ref, lnfs_ref, lnfb_ref, wout_ref, bout_ref, logits_ref):
    if True:
        x = zq_ref[...] + pos_ref[...]
        head_of_lane = jax.lax.broadcasted_iota(jnp.int32, (_S, _D), 1) // _DH

        for i in range(_NB):
            h = _layernorm(x, ln1s_ref[i], ln1b_ref[i])
            qkv = _dot(h, wqkv_ref[i], (((1,), (0,)))) + bqkv_ref[i][None, :]
            q = qkv[:, 0:_D]
            k = qkv[:, _D:2 * _D]
            v = qkv[:, 2 * _D:3 * _D]
            qm = jnp.concatenate(
                [jnp.where(head_of_lane == hh, q, 0.0) for hh in range(_H)],
                axis=0)
            al = _dot(qm, k, (((1,), (1,)))) * 0.25
            al = al - jnp.max(al, axis=1, keepdims=True)
            e = jnp.exp(al)
            p = e / jnp.sum(e, axis=1, keepdims=True)
            ost = _dot(p, v, (((1,), (0,))))
            o = jnp.zeros((_S, _D), jnp.float32)
            for hh in range(_H):
                o = o + jnp.where(head_of_lane == hh,
                                  ost[hh * _S:(hh + 1) * _S], 0.0)
            x = x + _dot(o, wo_ref[i], (((1,), (0,)))) + bo_ref[i][None, :]
            h2 = _layernorm(x, ln2s_ref[i], ln2b_ref[i])
            g = _dot(h2, w1_ref[i], (((1,), (0,)))) + b1_ref[i][None, :]
            x = x + _dot(_gelu_tanh(g), w2_ref[i], (((1,), (0,)))) \
                + b2_ref[i][None, :]

        xf = _layernorm(x, lnfs_ref[...], lnfb_ref[...])
        logits_ref[...] = _dot(xf, wout_ref[...], (((1,), (0,)))) \
            + bout_ref[...][None, :]


def sc_gather_body(cb_hbm, idx_hbm, out_hbm, idx_v, rows_v, sem):
    wid = lax.axis_index("s") * _NC + lax.axis_index("c")
    base = wid * _RPW
    pltpu.sync_copy(idx_hbm.at[pl.ds(base, _RPW)], idx_v)
    pltpu.async_copy(cb_hbm.at[idx_v], rows_v, sem).wait()
    pltpu.sync_copy(rows_v, out_hbm.at[pl.ds(base, _RPW)])


_sc_gather = functools.partial(
    pl.kernel,
    out_type=jax.ShapeDtypeStruct((_BS, _D), jnp.float32),
    mesh=plsc.VectorSubcoreMesh(core_axis_name="c", subcore_axis_name="s"),
    compiler_params=pltpu.CompilerParams(use_tc_tiling_on_sc=False),
    scratch_types=[
        pltpu.VMEM((_RPW,), jnp.int32),
        pltpu.VMEM((_RPW, _D), jnp.float32),
        pltpu.SemaphoreType.DMA,
    ],
)(sc_gather_body)


def _full(shape):
    return pl.BlockSpec(shape, lambda i: tuple(0 for _ in shape))


@jax.jit
def _run(zef, codebook, pos_emb, ln1_s, ln1_b, Wqkv, bqkv, Wo, bo, ln2_s,
         ln2_b, W1, b1, W2, b2, lnf_s, lnf_b, Wout, bout):
    zq_tc, idx = pl.pallas_call(
        vq_body,
        grid=(_NQ,),
        in_specs=[
            pl.BlockSpec((_TQ, _D), lambda i: (i, 0)),
            _full((_K, _D)),
        ],
        out_specs=(
            pl.BlockSpec((_TQ, _D), lambda i: (i, 0)),
            pl.BlockSpec((_TQ, 1), lambda i: (i, 0)),
        ),
        out_shape=(
            jax.ShapeDtypeStruct((_BS, _D), jnp.float32),
            jax.ShapeDtypeStruct((_BS, 1), jnp.int32),
        ),
    )(zef, codebook)

    zq = _sc_gather(codebook, idx.reshape(_BS))

    logits = pl.pallas_call(
        vit_body,
        grid=(_B,),
        in_specs=[
            pl.BlockSpec((_S, _D), lambda i: (i, 0)),
            _full((_S, _D)),
            _full((_NB, _D)), _full((_NB, _D)),
            _full((_NB, _D, 3 * _D)), _full((_NB, 3 * _D)),
            _full((_NB, _D, _D)), _full((_NB, _D)),
            _full((_NB, _D)), _full((_NB, _D)),
            _full((_NB, _D, _DFF)), _full((_NB, _DFF)),
            _full((_NB, _DFF, _D)), _full((_NB, _D)),
            _full((_D,)), _full((_D,)),
            _full((_D, _DP)), _full((_DP,)),
        ],
        out_specs=pl.BlockSpec((_S, _DP), lambda i: (i, 0)),
        out_shape=jax.ShapeDtypeStruct((_BS, _DP), jnp.float32),
    )(zq_tc, pos_emb, ln1_s, ln1_b, Wqkv, bqkv, Wo, bo, ln2_s, ln2_b,
      W1, b1, W2, b2, lnf_s, lnf_b, Wout, bout)
    return logits, zq


def kernel(ze, codebook, pos_emb, ln1_s, ln1_b, Wqkv, bqkv, Wo, bo, ln2_s,
           ln2_b, W1, b1, W2, b2, lnf_s, lnf_b, Wout, bout):
    zef = ze.reshape(_BS, _D)
    logits, zq = _run(zef, codebook, pos_emb, ln1_s, ln1_b, Wqkv, bqkv, Wo,
                      bo, ln2_s, ln2_b, W1, b1, W2, b2, lnf_s, lnf_b, Wout,
                      bout)
    return logits.reshape(_B, _S, _DP), zq.reshape(_B, _S, _D)
